# Initial kernel scaffold; baseline (speedup 1.0000x reference)
import jax
import jax.numpy as jnp
from jax.experimental import pallas as pl


def kernel(x, Wq, Wk, Wv):
    raise NotImplementedError("write your pallas kernel here")


# speedup vs baseline: 2.0409x; 2.0409x over previous
import functools
import math

import jax
import jax.numpy as jnp
from jax.experimental import pallas as pl
from jax.experimental.pallas import tpu as pltpu

N = 8192
C = 768
L = 2048
KC = 512
DL = 128


def _attn_seg_kernel(x_ref, wq_ref, wk_ref, wv_ref, num_ref, den_ref):
    xb = x_ref[...]
    wq = wq_ref[...]
    wk = wk_ref[...]
    wv = wv_ref[...]
    q = jnp.dot(xb, wq, preferred_element_type=jnp.float32)
    k = jnp.dot(xb, wk, preferred_element_type=jnp.float32)
    v = jnp.dot(xb, wv, preferred_element_type=jnp.float32)
    scale = 1.0 / math.sqrt(C)
    num = jnp.zeros((L, C), jnp.float32)
    den = jnp.zeros((L, 1), jnp.float32)
    for ci in range(L // KC):
        kcnk = k[ci * KC:(ci + 1) * KC, :]
        vcnk = v[ci * KC:(ci + 1) * KC, :]
        s = jax.lax.dot_general(
            q, kcnk, (((1,), (1,)), ((), ())),
            preferred_element_type=jnp.float32) * scale
        e = jnp.exp(s)
        den = den + jnp.sum(e, axis=1, keepdims=True)
        num = num + jnp.dot(e, vcnk, preferred_element_type=jnp.float32)
    num_ref[...] = num
    den_ref[...] = jnp.broadcast_to(den, (L, DL))


def _run_segs(xview, Wq, Wk, Wv, nseg):
    return pl.pallas_call(
        _attn_seg_kernel,
        grid=(nseg,),
        in_specs=[
            pl.BlockSpec((L, C), lambda j: (j, 0)),
            pl.BlockSpec((C, C), lambda j: (0, 0)),
            pl.BlockSpec((C, C), lambda j: (0, 0)),
            pl.BlockSpec((C, C), lambda j: (0, 0)),
        ],
        out_specs=[
            pl.BlockSpec((L, C), lambda j: (j, 0)),
            pl.BlockSpec((L, DL), lambda j: (j, 0)),
        ],
        out_shape=[
            jax.ShapeDtypeStruct((nseg * L, C), jnp.float32),
            jax.ShapeDtypeStruct((nseg * L, DL), jnp.float32),
        ],
    )(xview, Wq, Wk, Wv)


def _mix_kernel(n1_ref, n2_ref, n3_ref, d1_ref, d2_ref, d3_ref, out_ref):
    rows = n1_ref.shape[0]
    zc = jnp.zeros((rows, C), jnp.float32)
    zd = jnp.zeros((rows, DL), jnp.float32)
    n2 = n2_ref[...]
    n3 = n3_ref[...]
    ntot = n1_ref[...] + jnp.concatenate(
        [n2[:, :C], zc, n2[:, C:], zc], axis=1) + jnp.concatenate(
        [n3, zc, zc, zc], axis=1)
    d2 = d2_ref[...]
    d3 = d3_ref[...]
    dtot = d1_ref[...] + jnp.concatenate(
        [d2[:, :DL], zd, d2[:, DL:], zd], axis=1) + jnp.concatenate(
        [d3, zd, zd, zd], axis=1)
    pieces = [
        ntot[:, g * C:(g + 1) * C] / dtot[:, g * DL:g * DL + 1]
        for g in range(4)
    ]
    out_ref[...] = jnp.concatenate(pieces, axis=1)


def _mix(num1, den1, num2, den2, num3, den3):
    nq = N // 4
    rb = nq // 4
    out = pl.pallas_call(
        _mix_kernel,
        grid=(4,),
        in_specs=[
            pl.BlockSpec((rb, 4 * C), lambda j: (j, 0)),
            pl.BlockSpec((rb, 2 * C), lambda j: (j, 0)),
            pl.BlockSpec((rb, C), lambda j: (j, 0)),
            pl.BlockSpec((rb, 4 * DL), lambda j: (j, 0)),
            pl.BlockSpec((rb, 2 * DL), lambda j: (j, 0)),
            pl.BlockSpec((rb, DL), lambda j: (j, 0)),
        ],
        out_specs=pl.BlockSpec((rb, 4 * C), lambda j: (j, 0)),
        out_shape=jax.ShapeDtypeStruct((nq, 4 * C), jnp.float32),
    )(
        num1.reshape(nq, 4 * C),
        num2.reshape(nq, 2 * C),
        num3,
        den1.reshape(nq, 4 * DL),
        den2.reshape(nq, 2 * DL),
        den3,
    )
    return out


def kernel(x, Wq, Wk, Wv):
    x2d = x[0]
    num1, den1 = _run_segs(x2d, Wq, Wk, Wv, 4)
    num2, den2 = _run_segs(x2d.reshape(N // 2, 2 * C), Wq, Wk, Wv, 2)
    num3, den3 = _run_segs(x2d.reshape(N // 4, 4 * C), Wq, Wk, Wv, 1)
    out = _mix(num1, den1, num2, den2, num3, den3)
    return out.reshape(1, N, C)
